# Initial kernel scaffold; baseline (speedup 1.0000x reference)
import jax
import jax.numpy as jnp
from jax.experimental import pallas as pl


def kernel(items, emb_bag_inputs, offsets, input_lengths, emb_table):
    raise NotImplementedError("write your pallas kernel here")


# speedup vs baseline: 2.2075x; 2.2075x over previous
import functools

import jax
import jax.numpy as jnp
from jax import lax
from jax.experimental import pallas as pl
from jax.experimental.pallas import tpu as pltpu
from jax.experimental.pallas import tpu_sc as plsc


def kernel(items, emb_bag_inputs, offsets, input_lengths, emb_table):
    n_items = offsets.shape[0]
    L = emb_bag_inputs.shape[0] // n_items
    B = items.shape[0]
    V, D = emb_table.shape

    info = plsc.get_sparse_core_info()
    LN = info.num_lanes
    NW = info.num_cores * info.num_subcores
    b_per_w = B // NW
    E = b_per_w * L
    CH = 64
    n_ch = b_per_w // CH

    mesh = plsc.VectorSubcoreMesh(core_axis_name="c", subcore_axis_name="s")

    @functools.partial(
        pl.kernel,
        mesh=mesh,
        out_type=jax.ShapeDtypeStruct((B, D), jnp.float32),
        compiler_params=pltpu.CompilerParams(
            needs_layout_passes=False, use_tc_tiling_on_sc=False),
        scratch_types=[
            pltpu.VMEM((b_per_w,), jnp.int32),
            pltpu.VMEM((E,), jnp.int32),
            pltpu.VMEM((E,), jnp.int32),
            pltpu.VMEM((CH * L, D), jnp.float32),
            pltpu.VMEM((b_per_w, D), jnp.float32),
            pltpu.SemaphoreType.DMA,
        ],
    )
    def bag_kernel(items_hbm, bag_hbm, table_hbm, out_hbm,
                   items_v, eidx_v, feat_v, rows_v, out_v, sem):
        wid = lax.axis_index("s") * info.num_cores + lax.axis_index("c")
        base = wid * b_per_w
        pltpu.sync_copy(items_hbm.at[pl.ds(base, b_per_w)], items_v)

        lane = lax.iota(jnp.int32, LN)

        def expand_body(g, carry):
            iv = items_v[pl.ds(g * LN, LN)] * L
            pos = lane * L + g * (LN * L)
            for j in range(L):
                plsc.store_scatter(eidx_v, [pos + j], iv + j)
            return carry

        lax.fori_loop(0, b_per_w // LN, expand_body, 0)

        pltpu.async_copy(bag_hbm.at[eidx_v], feat_v, sem).wait()

        def chunk_body(c, carry):
            idx = feat_v.at[pl.ds(c * (CH * L), CH * L)]
            pltpu.async_copy(table_hbm.at[idx], rows_v, sem).wait()

            def item_body(i, carry2):
                for d in range(0, D, LN):
                    acc = rows_v[i * L, pl.ds(d, LN)]
                    for j in range(1, L):
                        acc = acc + rows_v[i * L + j, pl.ds(d, LN)]
                    out_v[c * CH + i, pl.ds(d, LN)] = acc
                return carry2

            lax.fori_loop(0, CH, item_body, 0)
            return carry

        lax.fori_loop(0, n_ch, chunk_body, 0)
        pltpu.sync_copy(out_v, out_hbm.at[pl.ds(base, b_per_w)])

    return bag_kernel(items, emb_bag_inputs, emb_table)


# speedup vs baseline: 2.5116x; 1.1377x over previous
import functools

import jax
import jax.numpy as jnp
from jax import lax
from jax.experimental import pallas as pl
from jax.experimental.pallas import tpu as pltpu
from jax.experimental.pallas import tpu_sc as plsc


def kernel(items, emb_bag_inputs, offsets, input_lengths, emb_table):
    n_items = offsets.shape[0]
    L = emb_bag_inputs.shape[0] // n_items
    B = items.shape[0]
    V, D = emb_table.shape

    info = plsc.get_sparse_core_info()
    LN = info.num_lanes
    NW = info.num_cores * info.num_subcores
    b_per_w = B // NW
    E = b_per_w * L
    CH = 128
    n_ch = b_per_w // CH

    mesh = plsc.VectorSubcoreMesh(core_axis_name="c", subcore_axis_name="s")

    @functools.partial(
        pl.kernel,
        mesh=mesh,
        out_type=jax.ShapeDtypeStruct((B, D), jnp.float32),
        compiler_params=pltpu.CompilerParams(
            needs_layout_passes=False, use_tc_tiling_on_sc=False),
        scratch_types=[
            pltpu.VMEM((b_per_w,), jnp.int32),
            pltpu.VMEM((E,), jnp.int32),
            pltpu.VMEM((E,), jnp.int32),
            pltpu.VMEM((CH * L, D), jnp.float32),
            pltpu.VMEM((CH * L, D), jnp.float32),
            pltpu.VMEM((b_per_w, D), jnp.float32),
            pltpu.SemaphoreType.DMA,
            pltpu.SemaphoreType.DMA,
        ],
    )
    def bag_kernel(items_hbm, bag_hbm, table_hbm, out_hbm,
                   items_v, eidx_v, feat_v, rows0_v, rows1_v, out_v,
                   sem0, sem1):
        wid = lax.axis_index("s") * info.num_cores + lax.axis_index("c")
        base = wid * b_per_w
        pltpu.sync_copy(items_hbm.at[pl.ds(base, b_per_w)], items_v)

        lane = lax.iota(jnp.int32, LN)

        def expand_body(g, carry):
            iv = items_v[pl.ds(g * LN, LN)] * L
            pos = lane * L + g * (LN * L)
            for j in range(L):
                plsc.store_scatter(eidx_v, [pos + j], iv + j)
            return carry

        lax.fori_loop(0, b_per_w // LN, expand_body, 0)

        pltpu.async_copy(bag_hbm.at[eidx_v], feat_v, sem0).wait()

        bufs = (rows0_v, rows1_v)
        sems = (sem0, sem1)

        def start(c):
            idx = feat_v.at[pl.ds(c * (CH * L), CH * L)]
            return pltpu.async_copy(table_hbm.at[idx], bufs[c % 2],
                                    sems[c % 2])

        pending = start(0)
        for c in range(n_ch):
            nxt = start(c + 1) if c + 1 < n_ch else None
            pending.wait()
            pending = nxt
            rows_v = bufs[c % 2]

            @plsc.parallel_loop(0, CH, 1, unroll=4)
            def item_body(i):
                for d in range(0, D, LN):
                    acc = rows_v[i * L, pl.ds(d, LN)]
                    for j in range(1, L):
                        acc = acc + rows_v[i * L + j, pl.ds(d, LN)]
                    out_v[c * CH + i, pl.ds(d, LN)] = acc

        pltpu.sync_copy(out_v, out_hbm.at[pl.ds(base, b_per_w)])

    return bag_kernel(items, emb_bag_inputs, emb_table)
